# baseline (speedup 1.0000x reference)
import functools

import jax
import jax.numpy as jnp
from jax import lax
from jax.experimental import pallas as pl
from jax.experimental.pallas import tpu as pltpu
from jax.experimental.pallas import tpu_sc as plsc

_EPS = 1e-06
_TINY = 1e-30
_NIMG = 4
_N = 4096
_D = 32
_SEG = 576
_GSEG = _NIMG * _SEG
_AUGW = 80
_NW = 32
_CHUNK = (_NIMG * _N) // _NW
_RPT = _GSEG // 16


def _prep_kernel(pred_ref, aug_ref):
    p = pred_ref[...]
    na2 = jnp.sum(p * p, axis=1, keepdims=True)
    phat = p * lax.rsqrt(jnp.maximum(na2, _TINY))
    onecol = (lax.broadcasted_iota(jnp.int32, (_NIMG * _N, 16), 1)
              == 0).astype(jnp.float32)
    aug_ref[...] = jnp.concatenate([p, phat, onecol], axis=1)


def _make_seg_call():
    mesh = plsc.VectorSubcoreMesh(core_axis_name="c", subcore_axis_name="s")

    @functools.partial(
        pl.kernel,
        mesh=mesh,
        out_type=jax.ShapeDtypeStruct((2, _GSEG, _AUGW), jnp.float32),
        scratch_types=[
            pltpu.VMEM((_CHUNK, _AUGW), jnp.float32),
            pltpu.VMEM((_CHUNK,), jnp.int32),
            pltpu.VMEM((_CHUNK,), jnp.int32),
            pltpu.VMEM((4, 128), jnp.int32),
            pltpu.VMEM((_RPT, _AUGW), jnp.float32),
            pltpu.VMEM_SHARED((_GSEG, _AUGW), jnp.float32),
        ],
    )
    def seg_kernel(aug_hbm, gt_hbm, an_hbm, out_hbm,
                   aug_v, gt_v, an_v, idx_v, zero_v, table):
        c = lax.axis_index("c")
        s = lax.axis_index("s")
        w = s * 2 + c
        base = w * _CHUNK
        img = w // (_N // _CHUNK)
        gbase = img * _SEG

        zrow = jnp.zeros((16,), jnp.float32)
        for r in range(_RPT):
            for kc in range(_AUGW // 16):
                zero_v[r, pl.ds(kc * 16, 16)] = zrow
        pltpu.sync_copy(zero_v, table.at[pl.ds(s * _RPT, _RPT)])

        pltpu.sync_copy(aug_hbm.at[pl.ds(base, _CHUNK)], aug_v)
        pltpu.sync_copy(gt_hbm.at[pl.ds(base, _CHUNK)], gt_v)
        pltpu.sync_copy(an_hbm.at[pl.ds(base, _CHUNK)], an_v)

        for k in range(_CHUNK // 16):
            g = gt_v[pl.ds(k * 16, 16)]
            a = an_v[pl.ds(k * 16, 16)]
            idx_v[k // 8, pl.ds((k % 8) * 16, 16)] = gbase + a * 64 + g

        plsc.subcore_barrier()

        for k in range(4):
            pltpu.sync_copy(aug_v.at[pl.ds(k * 128, 128)],
                            table.at[idx_v.at[k]], add=True)

        plsc.subcore_barrier()

        pltpu.sync_copy(table.at[pl.ds(s * _RPT, _RPT)],
                        out_hbm.at[c, pl.ds(s * _RPT, _RPT)])

    return seg_kernel


_seg_call = _make_seg_call()


def _finish_kernel(parts_ref, out_ref):
    s01 = parts_ref[0] + parts_ref[1]
    S = s01[:, 0:_D]
    P = s01[:, _D:2 * _D]
    cnt = s01[:, 2 * _D:2 * _D + 1]
    present = cnt > 0.0
    pf = present.astype(jnp.float32)
    safe = jnp.where(present, cnt, 1.0)
    S2 = jnp.sum(S * S, axis=1, keepdims=True)
    that = S * lax.rsqrt(jnp.maximum(S2, _TINY))
    pull_g = 1.0 - jnp.sum(that * P, axis=1, keepdims=True) / safe

    sel = (lax.broadcasted_iota(jnp.int32, (_NIMG * 9, _GSEG), 1) // 64
           == lax.broadcasted_iota(jnp.int32, (_NIMG * 9, _GSEG), 0)
           ).astype(jnp.float32)
    dn = (((1,), (0,)), ((), ()))
    obj = lax.dot_general(sel, pf, dn,
                          preferred_element_type=jnp.float32)
    pullnum = lax.dot_general(sel, pf * pull_g, dn,
                              preferred_element_type=jnp.float32)
    Sa = lax.dot_general(sel, pf * that, dn,
                         preferred_element_type=jnp.float32)
    els = lax.dot_general(sel, cnt, dn,
                          preferred_element_type=jnp.float32)

    Ssq = jnp.sum(Sa * Sa, axis=1, keepdims=True)
    push = (obj * obj + Ssq - 2.0 * obj) / (((obj - 1.0) * obj + _EPS) * 2.0)
    pull = pullnum / (obj + _EPS)
    la = jnp.where(obj <= 1.0, 0.0, pull + push)
    la = jnp.where(els > 0.0, la, 0.0)

    imgsel = (lax.broadcasted_iota(jnp.int32, (_NIMG, _NIMG * 9), 1) // 9
              == lax.broadcasted_iota(jnp.int32, (_NIMG, _NIMG * 9), 0)
              ).astype(jnp.float32)
    an_count = lax.dot_general(imgsel, (els > 0.0).astype(jnp.float32), dn,
                               preferred_element_type=jnp.float32)
    img_loss = lax.dot_general(imgsel, la, dn,
                               preferred_element_type=jnp.float32) / an_count
    out_ref[...] = jnp.full((1, 1), jnp.sum(img_loss) / _NIMG, jnp.float32)


def kernel(pred, gt_inds, anchor_inds):
    pred_flat = pred.reshape(_NIMG * _N, _D)
    gt_flat = gt_inds.astype(jnp.int32).reshape(-1)
    an_flat = anchor_inds.astype(jnp.int32).reshape(-1)

    aug = pl.pallas_call(
        _prep_kernel,
        out_shape=jax.ShapeDtypeStruct((_NIMG * _N, _AUGW), jnp.float32),
    )(pred_flat)

    parts = _seg_call(aug, gt_flat, an_flat)

    out = pl.pallas_call(
        _finish_kernel,
        out_shape=jax.ShapeDtypeStruct((1, 1), jnp.float32),
    )(parts)
    return out[0, 0]
